# baseline (speedup 1.0000x reference)
import numpy as np
import jax
import jax.numpy as jnp
from jax import lax
from jax.experimental import pallas as pl
from jax.experimental.pallas import tpu as pltpu

_NP = 6
_NSEG = 8
_NIN = 64
_NOUT = 64
_K = _NP * _NSEG
_KP = 64
_LEN = 2.0
_HALF = 1.0
_BT = 256

_X = np.linspace(-1.0, 1.0, _NP).astype(np.float32)
_INVD = np.array(
    [1.0 / np.prod([_X[j] - _X[m] for m in range(_NP) if m != j])
     for j in range(_NP)],
    dtype=np.float32,
)
_CR = np.array(
    [[_X[m] for m in range(_NP) if m != j] for j in range(_NP)],
    dtype=np.float32,
).T


def _body(x_ref, wp_ref, sw_ref, pw_ref, o_ref, wv_ref, sem):
    @pl.when(pl.program_id(0) == 0)
    def _start():
        pltpu.make_async_copy(wp_ref, wv_ref, sem).start()

    xv = x_ref[...].T

    idm = ((xv + _HALF) / _LEN * _NSEG).astype(jnp.int32)
    idm = jnp.minimum(idm, _NSEG - 1)
    idm = jnp.maximum(idm, 0)
    idf = idm.astype(jnp.float32)
    x_min = idf / _NSEG * 2.0 - 1.0
    x_max = (idf + 1.0) / _NSEG * 2.0 - 1.0
    x_in = _LEN * ((xv - x_min) / (x_max - x_min)) - _HALF

    jidx = lax.broadcasted_iota(jnp.int32, (1, _NSEG, 1), 1)
    invd = jnp.zeros((1, _NSEG, 1), jnp.float32)
    for j in range(_NP):
        invd = jnp.where(jidx == j, float(_INVD[j]), invd)
    crs = []
    for r in range(_NP - 1):
        c = jnp.zeros((1, _NSEG, 1), jnp.float32)
        for j in range(_NP):
            c = jnp.where(jidx == j, float(_CR[r, j]), c)
        crs.append(c)
    x6 = jnp.broadcast_to(x_in[:, None, :], (_NIN, _NSEG, _BT))
    t0 = x6 - crs[0]
    t1 = x6 - crs[1]
    t2 = x6 - crs[2]
    t3 = x6 - crs[3]
    t4 = x6 - crs[4]
    bas = ((t0 * t1) * (t2 * t3)) * (t4 * invd)

    m8 = jnp.where(idm[:, None, :] == jidx, 1.0, 0.0)

    bx = jnp.repeat(bas, _NSEG, axis=1)
    mx = jnp.concatenate([m8] * _NSEG, axis=1)
    coeff = bx * mx

    @pl.when(pl.program_id(0) == 0)
    def _wait():
        pltpu.make_async_copy(wp_ref, wv_ref, sem).wait()

    sum_acc = jnp.zeros((_NOUT, _BT), jnp.float32)
    prod_acc = jnp.full((_NOUT, _BT), 1.0, jnp.float32)
    for g in range(_NIN // 2):
        a0 = lax.dot_general(
            wv_ref[g, :, 0:_KP], coeff[2 * g],
            (((1,), (0,)), ((), ())),
            preferred_element_type=jnp.float32,
        )
        a1 = lax.dot_general(
            wv_ref[g, :, _KP:2 * _KP], coeff[2 * g + 1],
            (((1,), (0,)), ((), ())),
            preferred_element_type=jnp.float32,
        )
        sum_acc = sum_acc + (a0 + a1)
        prod_acc = prod_acc * (a0 * a1)

    o_ref[...] = sum_acc.T * sw_ref[...] + prod_acc.T * pw_ref[...]


def kernel(x, w, sum_w, prod_w):
    batch = x.shape[0]
    wtj = jnp.transpose(w.reshape(_NOUT, _NIN, _NSEG, _NP), (1, 0, 3, 2))
    wtj = jnp.pad(wtj, ((0, 0), (0, 0), (0, _NSEG - _NP), (0, 0)))
    wp = jnp.transpose(
        wtj.reshape(_NIN // 2, 2, _NOUT, _KP), (0, 2, 1, 3)
    ).reshape(_NIN // 2, _NOUT, 2 * _KP)
    return pl.pallas_call(
        _body,
        grid=(batch // _BT,),
        in_specs=[
            pl.BlockSpec((_BT, _NIN), lambda t: (t, 0)),
            pl.BlockSpec(memory_space=pl.ANY),
            pl.BlockSpec((1, _NOUT), lambda t: (0, 0)),
            pl.BlockSpec((1, _NOUT), lambda t: (0, 0)),
        ],
        out_specs=pl.BlockSpec((_BT, _NOUT), lambda t: (t, 0)),
        out_shape=jax.ShapeDtypeStruct((batch, _NOUT), jnp.float32),
        scratch_shapes=[
            pltpu.VMEM((_NIN // 2, _NOUT, 2 * _KP), jnp.float32),
            pltpu.SemaphoreType.DMA,
        ],
    )(x, wp, sum_w.reshape(1, _NOUT), prod_w.reshape(1, _NOUT))


# speedup vs baseline: 1.1893x; 1.1893x over previous
import numpy as np
import jax
import jax.numpy as jnp
from jax import lax
from jax.experimental import pallas as pl

_NP = 6
_NSEG = 8
_NIN = 64
_NOUT = 64
_K = _NP * _NSEG
_KP = 64
_LEN = 2.0
_HALF = 1.0
_BT = 256

_X = np.linspace(-1.0, 1.0, _NP).astype(np.float32)
_INVD = np.array(
    [1.0 / np.prod([_X[j] - _X[m] for m in range(_NP) if m != j])
     for j in range(_NP)],
    dtype=np.float32,
)
_CR = np.array(
    [[_X[m] for m in range(_NP) if m != j] for j in range(_NP)],
    dtype=np.float32,
).T


def _body(x_ref, wp_ref, sw_ref, pw_ref, o_ref):
    xv = x_ref[...].T

    idm = ((xv + _HALF) / _LEN * _NSEG).astype(jnp.int32)
    idm = jnp.minimum(idm, _NSEG - 1)
    idm = jnp.maximum(idm, 0)
    idf = idm.astype(jnp.float32)
    x_min = idf / _NSEG * 2.0 - 1.0
    x_max = (idf + 1.0) / _NSEG * 2.0 - 1.0
    x_in = _LEN * ((xv - x_min) / (x_max - x_min)) - _HALF

    jidx = lax.broadcasted_iota(jnp.int32, (1, _NSEG, 1), 1)
    invd = jnp.zeros((1, _NSEG, 1), jnp.float32)
    for j in range(_NP):
        invd = jnp.where(jidx == j, float(_INVD[j]), invd)
    crs = []
    for r in range(_NP - 1):
        c = jnp.zeros((1, _NSEG, 1), jnp.float32)
        for j in range(_NP):
            c = jnp.where(jidx == j, float(_CR[r, j]), c)
        crs.append(c)
    x6 = jnp.broadcast_to(x_in[:, None, :], (_NIN, _NSEG, _BT))
    t0 = x6 - crs[0]
    t1 = x6 - crs[1]
    t2 = x6 - crs[2]
    t3 = x6 - crs[3]
    t4 = x6 - crs[4]
    bas = ((t0 * t1) * (t2 * t3)) * (t4 * invd)

    m8 = jnp.where(idm[:, None, :] == jidx, 1.0, 0.0)

    bx = jnp.repeat(bas, _NSEG, axis=1)
    mx = jnp.concatenate([m8] * _NSEG, axis=1)
    coeff = bx * mx

    sum_acc = jnp.zeros((_NOUT, _BT), jnp.float32)
    prod_acc = jnp.full((_NOUT, _BT), 1.0, jnp.float32)
    for g in range(_NIN // 2):
        a0 = lax.dot_general(
            wp_ref[g, :, 0:_KP], coeff[2 * g],
            (((1,), (0,)), ((), ())),
            preferred_element_type=jnp.float32,
        )
        a1 = lax.dot_general(
            wp_ref[g, :, _KP:2 * _KP], coeff[2 * g + 1],
            (((1,), (0,)), ((), ())),
            preferred_element_type=jnp.float32,
        )
        sum_acc = sum_acc + (a0 + a1)
        prod_acc = prod_acc * (a0 * a1)

    o_ref[...] = sum_acc.T * sw_ref[...] + prod_acc.T * pw_ref[...]


def kernel(x, w, sum_w, prod_w):
    batch = x.shape[0]
    wtj = jnp.transpose(w.reshape(_NOUT, _NIN, _NSEG, _NP), (1, 0, 3, 2))
    wtj = jnp.pad(wtj, ((0, 0), (0, 0), (0, _NSEG - _NP), (0, 0)))
    wp = jnp.transpose(
        wtj.reshape(_NIN // 2, 2, _NOUT, _KP), (0, 2, 1, 3)
    ).reshape(_NIN // 2, _NOUT, 2 * _KP)
    return pl.pallas_call(
        _body,
        grid=(batch // _BT,),
        in_specs=[
            pl.BlockSpec((_BT, _NIN), lambda t: (t, 0)),
            pl.BlockSpec((_NIN // 2, _NOUT, 2 * _KP), lambda t: (0, 0, 0)),
            pl.BlockSpec((1, _NOUT), lambda t: (0, 0)),
            pl.BlockSpec((1, _NOUT), lambda t: (0, 0)),
        ],
        out_specs=pl.BlockSpec((_BT, _NOUT), lambda t: (t, 0)),
        out_shape=jax.ShapeDtypeStruct((batch, _NOUT), jnp.float32),
    )(x, wp, sum_w.reshape(1, _NOUT), prod_w.reshape(1, _NOUT))
